# Initial kernel scaffold; baseline (speedup 1.0000x reference)
import jax
import jax.numpy as jnp
from jax.experimental import pallas as pl


def kernel(atom_feats, bond_feats, global_feats, Wa1, ba1, Wa2, ba2, Wa3, ba3, Wb1, bb1, Wb2, bb2, Wb3, bb3, Wg1, bg1, Wg2, bg2, Wg3, bg3, bond_atoms, atom_graph, bond_graph):
    raise NotImplementedError("write your pallas kernel here")


# speedup vs baseline: 1.1782x; 1.1782x over previous
import functools
import jax
import jax.numpy as jnp
from jax.experimental import pallas as pl
from jax.experimental.pallas import tpu as pltpu

D = 32


def _softplus(x):
    return jnp.maximum(x, 0.0) + jnp.log1p(jnp.exp(-jnp.abs(x)))


def _mlp3(x, W1, b1, W2, b2, W3, b3):
    h = _softplus(jnp.dot(x, W1, preferred_element_type=jnp.float32) + b1)
    h = _softplus(jnp.dot(h, W2, preferred_element_type=jnp.float32) + b2)
    return jnp.dot(h, W3, preferred_element_type=jnp.float32) + b3


def _atom_kernel(af, sums, cnt, mg, W1, b1, W2, b2, W3, b3, out):
    c = jnp.maximum(cnt[...], 1.0)
    mb = sums[...] / c
    x = jnp.concatenate([af[...], mb, mg[...]], axis=1)
    out[...] = _mlp3(x, W1[...], b1[...], W2[...], b2[...], W3[...], b3[...])


def _atom_mlp(af, sums, cnt, mg, W1, b1, W2, b2, W3, b3):
    n = af.shape[0]
    B = 2000
    grid = (n // B,)
    row = lambda i: (i, 0)
    fixed = lambda i: (0, 0)
    return pl.pallas_call(
        _atom_kernel,
        grid=grid,
        in_specs=[
            pl.BlockSpec((B, D), row),
            pl.BlockSpec((B, D), row),
            pl.BlockSpec((B, 1), row),
            pl.BlockSpec((B, D), row),
            pl.BlockSpec(W1.shape, fixed),
            pl.BlockSpec(b1.shape, fixed),
            pl.BlockSpec(W2.shape, fixed),
            pl.BlockSpec(b2.shape, fixed),
            pl.BlockSpec(W3.shape, fixed),
            pl.BlockSpec(b3.shape, fixed),
        ],
        out_specs=pl.BlockSpec((B, D), row),
        out_shape=jax.ShapeDtypeStruct((n, D), jnp.float32),
    )(af, sums, cnt, mg, W1, b1, W2, b2, W3, b3)


def _bond_kernel(bf, a0, a1, gb, W1, b1, W2, b2, W3, b3, out):
    x = jnp.concatenate([bf[...], a0[...], a1[...], gb[...]], axis=1)
    out[...] = _mlp3(x, W1[...], b1[...], W2[...], b2[...], W3[...], b3[...])


def _bond_mlp(bf, a0, a1, gb, W1, b1, W2, b2, W3, b3):
    n = bf.shape[0]
    B = 3200
    grid = (n // B,)
    row = lambda i: (i, 0)
    fixed = lambda i: (0, 0)
    return pl.pallas_call(
        _bond_kernel,
        grid=grid,
        in_specs=[
            pl.BlockSpec((B, D), row),
            pl.BlockSpec((B, D), row),
            pl.BlockSpec((B, D), row),
            pl.BlockSpec((B, D), row),
            pl.BlockSpec(W1.shape, fixed),
            pl.BlockSpec(b1.shape, fixed),
            pl.BlockSpec(W2.shape, fixed),
            pl.BlockSpec(b2.shape, fixed),
            pl.BlockSpec(W3.shape, fixed),
            pl.BlockSpec(b3.shape, fixed),
        ],
        out_specs=pl.BlockSpec((B, D), row),
        out_shape=jax.ShapeDtypeStruct((n, D), jnp.float32),
    )(bf, a0, a1, gb, W1, b1, W2, b2, W3, b3)


def _glob_kernel(gf, sa, ca, sb, cb, W1, b1, W2, b2, W3, b3, out):
    ma = sa[...] / jnp.maximum(ca[...], 1.0)
    mb = sb[...] / jnp.maximum(cb[...], 1.0)
    x = jnp.concatenate([gf[...], ma, mb], axis=1)
    out[...] = _mlp3(x, W1[...], b1[...], W2[...], b2[...], W3[...], b3[...])


def _glob_mlp(gf, sa, ca, sb, cb, W1, b1, W2, b2, W3, b3):
    n = gf.shape[0]
    return pl.pallas_call(
        _glob_kernel,
        out_shape=jax.ShapeDtypeStruct((n, D), jnp.float32),
    )(gf, sa, ca, sb, cb, W1, b1, W2, b2, W3, b3)


def kernel(atom_feats, bond_feats, global_feats,
           Wa1, ba1, Wa2, ba2, Wa3, ba3,
           Wb1, bb1, Wb2, bb2, Wb3, bb3,
           Wg1, bg1, Wg2, bg2, Wg3, bg3,
           bond_atoms, atom_graph, bond_graph):
    n_atom = atom_feats.shape[0]
    n_bond = bond_feats.shape[0]
    n_graph = global_feats.shape[0]

    ba1r, ba2r, ba3r = ba1[None, :], ba2[None, :], ba3[None, :]
    bb1r, bb2r, bb3r = bb1[None, :], bb2[None, :], bb3[None, :]
    bg1r, bg2r, bg3r = bg1[None, :], bg2[None, :], bg3[None, :]

    idx0 = bond_atoms[:, 0]
    idx1 = bond_atoms[:, 1]

    sums = jnp.zeros((n_atom, D), jnp.float32).at[idx0].add(bond_feats)
    sums = sums.at[idx1].add(bond_feats)
    cnt = jnp.zeros((n_atom,), jnp.float32).at[idx0].add(1.0).at[idx1].add(1.0)
    mg = global_feats[atom_graph]

    atom_new = _atom_mlp(atom_feats, sums, cnt[:, None], mg,
                         Wa1, ba1r, Wa2, ba2r, Wa3, ba3r)

    a0 = atom_new[idx0]
    a1 = atom_new[idx1]
    gb = global_feats[bond_graph]

    bond_new = _bond_mlp(bond_feats, a0, a1, gb,
                         Wb1, bb1r, Wb2, bb2r, Wb3, bb3r)

    sa = jax.ops.segment_sum(atom_new, atom_graph, num_segments=n_graph)
    ca = jax.ops.segment_sum(jnp.ones((n_atom,), jnp.float32), atom_graph,
                             num_segments=n_graph)
    sb = jax.ops.segment_sum(bond_new, bond_graph, num_segments=n_graph)
    cb = jax.ops.segment_sum(jnp.ones((n_bond,), jnp.float32), bond_graph,
                             num_segments=n_graph)

    glob_new = _glob_mlp(global_feats, sa, ca[:, None], sb, cb[:, None],
                         Wg1, bg1r, Wg2, bg2r, Wg3, bg3r)
    return atom_new, bond_new, glob_new
